# baseline (speedup 1.0000x reference)
import functools

import jax
import jax.numpy as jnp
from jax import lax
from jax.experimental import pallas as pl
from jax.experimental.pallas import tpu as pltpu
from jax.experimental.pallas import tpu_sc as plsc

_C = 128
_LANES = 16
_NC = 2
_NS = 16
_NW = _NC * _NS
_R = 128
_NBUF = 4


def _build(n):
    rpw = n // _NW
    nchunk = rpw // _R
    tile_words = _R * _C

    mesh = plsc.VectorSubcoreMesh(core_axis_name="c", subcore_axis_name="s")

    @functools.partial(
        pl.kernel,
        out_type=jax.ShapeDtypeStruct((n * _C,), jnp.float32),
        mesh=mesh,
        compiler_params=pltpu.CompilerParams(needs_layout_passes=False),
        scratch_types=[
            pltpu.VMEM((rpw,), jnp.float32),
            pltpu.VMEM((_NBUF * tile_words,), jnp.float32),
            [pltpu.SemaphoreType.DMA] * _NBUF,
            pltpu.SemaphoreType.DMA,
        ],
    )
    def run(x_hbm, out_hbm, lab_v, tiles, sems, sem_lab):
        wid = lax.axis_index("s") * _NC + lax.axis_index("c")
        base = wid * rpw

        lab_cp = pltpu.make_async_copy(x_hbm.at[pl.ds(base, rpw)], lab_v,
                                       sem_lab)
        lab_cp.start()

        zeros16 = jnp.zeros((_LANES,), jnp.float32)
        zunroll = 16

        def zbody(i, _):
            for u in range(zunroll):
                tiles[pl.ds((i * zunroll + u) * _LANES, _LANES)] = zeros16
            return 0

        lax.fori_loop(0, (_NBUF * tile_words) // (_LANES * zunroll), zbody, 0)
        lab_cp.wait()

        iota = lax.iota(jnp.int32, _LANES)
        fives = jnp.full((_LANES,), 5.0, jnp.float32)

        def scatter_chunk(kk, b, val):
            def jbody(j, _):
                lv = lab_v[pl.ds(kk * _R + j * _LANES, _LANES)]
                col = jnp.clip(lv.astype(jnp.int32), 0, _C - 1)
                idx = (b * tile_words + j * (_LANES * _C)) + iota * _C + col
                plsc.store_scatter(tiles, [idx], val)
                return 0

            lax.fori_loop(0, _R // _LANES, jbody, 0)

        def dma(kk, b):
            src = tiles.at[pl.ds(b * tile_words, tile_words)]
            dst = out_hbm.at[pl.ds(base * _C + kk * tile_words, tile_words)]
            return pltpu.make_async_copy(src, dst, sems[b])

        for b in range(_NBUF):
            scatter_chunk(b, b, fives)
            dma(b, b).start()

        def loop_body(i, _):
            k0 = _NBUF * i
            for b in range(_NBUF):
                kk = k0 + b
                dma(kk - _NBUF, b).wait()
                scatter_chunk(kk - _NBUF, b, zeros16)
                scatter_chunk(kk, b, fives)
                dma(kk, b).start()
            return 0

        lax.fori_loop(1, nchunk // _NBUF, loop_body, 0)

        for b in range(_NBUF):
            dma(nchunk - _NBUF + b, b).wait()

    return run


def kernel(x):
    n = x.shape[0]
    out_flat = _build(n)(x.reshape(-1))
    return out_flat.reshape(n, _C)


# speedup vs baseline: 1.0254x; 1.0254x over previous
import functools

import jax
import jax.numpy as jnp
from jax import lax
from jax.experimental import pallas as pl
from jax.experimental.pallas import tpu as pltpu
from jax.experimental.pallas import tpu_sc as plsc

_C = 128
_LANES = 16
_NC = 2
_NS = 16
_NW = _NC * _NS
_R = 256
_NBUF = 2


def _build(n):
    rpw = n // _NW
    nchunk = rpw // _R
    tile_words = _R * _C

    mesh = plsc.VectorSubcoreMesh(core_axis_name="c", subcore_axis_name="s")

    @functools.partial(
        pl.kernel,
        out_type=jax.ShapeDtypeStruct((n * _C,), jnp.float32),
        mesh=mesh,
        compiler_params=pltpu.CompilerParams(needs_layout_passes=False),
        scratch_types=[
            pltpu.VMEM((rpw,), jnp.float32),
            pltpu.VMEM((_NBUF * tile_words,), jnp.float32),
            [pltpu.SemaphoreType.DMA] * _NBUF,
            pltpu.SemaphoreType.DMA,
        ],
    )
    def run(x_hbm, out_hbm, lab_v, tiles, sems, sem_lab):
        wid = lax.axis_index("s") * _NC + lax.axis_index("c")
        base = wid * rpw

        lab_cp = pltpu.make_async_copy(x_hbm.at[pl.ds(base, rpw)], lab_v,
                                       sem_lab)
        lab_cp.start()

        zeros16 = jnp.zeros((_LANES,), jnp.float32)
        zunroll = 16

        def zero_buf(b):
            def zbody(i, _):
                for u in range(zunroll):
                    tiles[pl.ds(b * tile_words
                                + (i * zunroll + u) * _LANES, _LANES)] = zeros16
                return 0

            lax.fori_loop(0, tile_words // (_LANES * zunroll), zbody, 0)

        iota = lax.iota(jnp.int32, _LANES)
        fives = jnp.full((_LANES,), 5.0, jnp.float32)

        def scatter_chunk(kk, b, val):
            def jbody(j, _):
                lv = lab_v[pl.ds(kk * _R + j * _LANES, _LANES)]
                col = jnp.clip(lv.astype(jnp.int32), 0, _C - 1)
                idx = (b * tile_words + j * (_LANES * _C)) + iota * _C + col
                plsc.store_scatter(tiles, [idx], val)
                return 0

            lax.fori_loop(0, _R // _LANES, jbody, 0)

        def dma(kk, b):
            src = tiles.at[pl.ds(b * tile_words, tile_words)]
            dst = out_hbm.at[pl.ds(base * _C + kk * tile_words, tile_words)]
            return pltpu.make_async_copy(src, dst, sems[b])

        zero_buf(0)
        lab_cp.wait()
        for b in range(_NBUF):
            scatter_chunk(b, b, fives)
            dma(b, b).start()
            if b + 1 < _NBUF:
                zero_buf(b + 1)

        def loop_body(i, _):
            k0 = _NBUF * i
            for b in range(_NBUF):
                kk = k0 + b
                dma(kk - _NBUF, b).wait()
                scatter_chunk(kk - _NBUF, b, zeros16)
                scatter_chunk(kk, b, fives)
                dma(kk, b).start()
            return 0

        lax.fori_loop(1, nchunk // _NBUF, loop_body, 0)

        for b in range(_NBUF):
            dma(nchunk - _NBUF + b, b).wait()

    return run


def kernel(x):
    n = x.shape[0]
    out_flat = _build(n)(x.reshape(-1))
    return out_flat.reshape(n, _C)


# speedup vs baseline: 1.0400x; 1.0143x over previous
import functools

import jax
import jax.numpy as jnp
from jax import lax
from jax.experimental import pallas as pl
from jax.experimental.pallas import tpu as pltpu
from jax.experimental.pallas import tpu_sc as plsc

_C = 128
_LANES = 16
_NC = 2
_NS = 16
_NW = _NC * _NS
_R = 256
_NBUF = 2


def _build(n):
    rpw = n // _NW
    nchunk = rpw // _R
    tile_words = _R * _C

    mesh = plsc.VectorSubcoreMesh(core_axis_name="c", subcore_axis_name="s")

    @functools.partial(
        pl.kernel,
        out_type=jax.ShapeDtypeStruct((n * _C,), jnp.float32),
        mesh=mesh,
        compiler_params=pltpu.CompilerParams(needs_layout_passes=False),
        scratch_types=[
            pltpu.VMEM((rpw,), jnp.float32),
            pltpu.VMEM((_NBUF * tile_words,), jnp.float32),
            [pltpu.SemaphoreType.DMA] * _NBUF,
            pltpu.SemaphoreType.DMA,
        ],
    )
    def run(x_hbm, out_hbm, lab_v, tiles, sems, sem_lab):
        wid = lax.axis_index("s") * _NC + lax.axis_index("c")
        base = wid * rpw

        lab_cp = pltpu.make_async_copy(x_hbm.at[pl.ds(base, rpw)], lab_v,
                                       sem_lab)
        lab_cp.start()

        zeros16 = jnp.zeros((_LANES,), jnp.float32)
        zunroll = 16

        def zbody(i, _):
            for u in range(zunroll):
                tiles[pl.ds((i * zunroll + u) * _LANES, _LANES)] = zeros16
            return 0

        lax.fori_loop(0, (_NBUF * tile_words) // (_LANES * zunroll), zbody, 0)
        lab_cp.wait()

        iota = lax.iota(jnp.int32, _LANES)
        fives = jnp.full((_LANES,), 5.0, jnp.float32)

        def scatter_chunk(kk, b, val):
            def jbody(j, _):
                lv = lab_v[pl.ds(kk * _R + j * _LANES, _LANES)]
                col = jnp.clip(lv.astype(jnp.int32), 0, _C - 1)
                idx = (b * tile_words + j * (_LANES * _C)) + iota * _C + col
                plsc.store_scatter(tiles, [idx], val)
                return 0

            lax.fori_loop(0, _R // _LANES, jbody, 0)

        def dma(kk, b):
            src = tiles.at[pl.ds(b * tile_words, tile_words)]
            dst = out_hbm.at[pl.ds(base * _C + kk * tile_words, tile_words)]
            return pltpu.make_async_copy(src, dst, sems[b])

        for b in range(_NBUF):
            scatter_chunk(b, b, fives)
            dma(b, b).start()

        def loop_body(i, _):
            k0 = _NBUF * i
            for b in range(_NBUF):
                kk = k0 + b
                dma(kk - _NBUF, b).wait()
                scatter_chunk(kk - _NBUF, b, zeros16)
                scatter_chunk(kk, b, fives)
                dma(kk, b).start()
            return 0

        lax.fori_loop(1, nchunk // _NBUF, loop_body, 0)

        for b in range(_NBUF):
            dma(nchunk - _NBUF + b, b).wait()

    return run


def kernel(x):
    n = x.shape[0]
    out_flat = _build(n)(x.reshape(-1))
    return out_flat.reshape(n, _C)
